# baseline (speedup 1.0000x reference)
import functools

import jax
import jax.numpy as jnp
from jax import lax
from jax.experimental import pallas as pl
from jax.experimental.pallas import tpu as pltpu
from jax.experimental.pallas import tpu_sc as plsc

N = 10000
E = 320000
D = 128
H = 4
OUT = 128
C = 460

NC = 2
NS = 16
NH = N // NC
NHP = NH + 8
EPT = E // NS
EB = 80
NBATCH = EPT // EB
DW = 16


_BN = 2000


def _proj_body(x_ref, wl_ref, bl_ref, wr_ref, br_ref, ol_ref, or_ref):
    xb = x_ref[...]
    dn = (((1,), (1,)), ((), ()))
    l = lax.dot_general(xb, wl_ref[...], dn, preferred_element_type=jnp.float32)
    r = lax.dot_general(xb, wr_ref[...], dn, preferred_element_type=jnp.float32)
    ol_ref[...] = (l + bl_ref[0])[None]
    or_ref[...] = (r + br_ref[0])[None]


def _project(x, W_l, b_l2, W_r, b_r2):
    return pl.pallas_call(
        _proj_body,
        grid=(H, N // _BN),
        in_specs=[
            pl.BlockSpec((_BN, D), lambda h, n: (n, 0)),
            pl.BlockSpec((OUT, D), lambda h, n: (h, 0)),
            pl.BlockSpec((1, 1, OUT), lambda h, n: (h, 0, 0)),
            pl.BlockSpec((OUT, D), lambda h, n: (h, 0)),
            pl.BlockSpec((1, 1, OUT), lambda h, n: (h, 0, 0)),
        ],
        out_specs=[
            pl.BlockSpec((1, _BN, OUT), lambda h, n: (h, n, 0)),
            pl.BlockSpec((1, _BN, OUT), lambda h, n: (h, n, 0)),
        ],
        out_shape=[
            jax.ShapeDtypeStruct((H, N, OUT), jnp.float32),
            jax.ShapeDtypeStruct((H, N, OUT), jnp.float32),
        ],
    )(x, W_l, b_l2, W_r, b_r2)


_sc_mesh = plsc.VectorSubcoreMesh(core_axis_name="c", subcore_axis_name="s")


@functools.partial(
    pl.kernel,
    out_type=[
        jax.ShapeDtypeStruct((H, N, OUT), jnp.float32),
        jax.ShapeDtypeStruct((H, N, OUT), jnp.float32),
    ],
    mesh=_sc_mesh,
    scratch_types=[
        pltpu.VMEM((EB,), jnp.int32),
        pltpu.VMEM((EB,), jnp.int32),
        pltpu.VMEM((EB,), jnp.int32),
        pltpu.VMEM((EB, OUT), jnp.float32),
        pltpu.VMEM((EB, OUT), jnp.float32),
        pltpu.VMEM((EB, OUT), jnp.float32),
        pltpu.VMEM((H, OUT), jnp.float32),
        pltpu.SemaphoreType.DMA,
        pltpu.SemaphoreType.DMA,
        pltpu.VMEM_SHARED((NHP, OUT), jnp.float32),
        pltpu.VMEM_SHARED((NHP, OUT), jnp.float32),
    ],
)
def _sc_edges(src_hbm, dst_hbm, att_hbm, zacc_hbm, zden_hbm,
              xl0, xl1, xl2, xl3, xr0, xr1, xr2, xr3,
              acc_out, den_out,
              srcv, dstv, dstv2, lrows, rrows, wrow, attv,
              sem0, sem1, acc_s, den_s):
    c = lax.axis_index("c")
    s = lax.axis_index("s")
    cbase = c * NH

    pltpu.sync_copy(att_hbm, attv)

    xls = (xl0, xl1, xl2, xl3)
    xrs = (xr0, xr1, xr2, xr3)

    for h in range(H):
        pltpu.sync_copy(zacc_hbm, acc_s)
        pltpu.sync_copy(zden_hbm, den_s)
        plsc.subcore_barrier()

        def _batch_body(b, carry, h=h):
            base = s * EPT + b * EB
            pltpu.sync_copy(src_hbm.at[pl.ds(base, EB)], srcv)
            pltpu.sync_copy(dst_hbm.at[pl.ds(base, EB)], dstv)
            cp_l = pltpu.async_copy(xls[h].at[srcv], lrows, sem0)
            cp_r = pltpu.async_copy(xrs[h].at[dstv], rrows, sem1)
            cp_l.wait()
            cp_r.wait()

            for g in range(EB // 16):
                dv = dstv[pl.ds(g * 16, 16)]
                local = dv - cbase
                oob = (local < 0) | (local >= NH)
                dstv2[pl.ds(g * 16, 16)] = jnp.where(oob, NH, local)

            def _edge_body(i, ecarry, h=h):
                for e in (2 * i, 2 * i + 1):
                    acc = jnp.zeros((16,), jnp.float32)
                    for k in range(OUT // 16):
                        lv = lrows[e, pl.ds(k * 16, 16)]
                        rv = rrows[e, pl.ds(k * 16, 16)]
                        v = lv + rv
                        lr = jnp.maximum(v, 0.2 * v)
                        acc = acc + attv[h, pl.ds(k * 16, 16)] * lr
                    lane = lax.iota(jnp.int32, 16)
                    for sh in (1, 2, 4, 8):
                        acc = acc + acc.at[lane ^ sh].get(mode="promise_in_bounds")
                    wv = jnp.exp(acc)
                    for k in range(OUT // 16):
                        wrow[e, pl.ds(k * 16, 16)] = wv
                    for k in range(OUT // 16):
                        rrows[e, pl.ds(k * 16, 16)] = lrows[e, pl.ds(k * 16, 16)] * wv
                return ecarry

            lax.fori_loop(0, EB // 2, _edge_body, 0)
            pltpu.sync_copy(rrows, acc_s.at[dstv2], add=True)
            pltpu.sync_copy(wrow, den_s.at[dstv2], add=True)
            return carry

        lax.fori_loop(0, NBATCH, _batch_body, 0)
        plsc.subcore_barrier()

        pltpu.sync_copy(acc_s.at[pl.ds(0, NH)],
                        acc_out.at[h, pl.ds(cbase, NH)])
        pltpu.sync_copy(den_s.at[pl.ds(0, NH)],
                        den_out.at[h, pl.ds(cbase, NH)])
        plsc.subcore_barrier()


_BNF = 1000


def _final_body(acc_ref, den_ref, bias_ref, wfc_ref, bfc_ref, out_ref):
    hsum = jnp.zeros((_BNF, OUT), jnp.float32)
    for h in range(H):
        a = acc_ref[h]
        dsum = den_ref[h]
        hsum = hsum + a / (dsum[:, 0:1] + 1e-16)
    hmean = hsum * (1.0 / H) + bias_ref[...]
    dn = (((1,), (1,)), ((), ()))
    out = lax.dot_general(hmean, wfc_ref[...], dn,
                          preferred_element_type=jnp.float32)
    out_ref[...] = out + bfc_ref[...]


def _finalize(acc, den, bias2, W_fc, b_fc2):
    return pl.pallas_call(
        _final_body,
        grid=(N // _BNF,),
        in_specs=[
            pl.BlockSpec((H, _BNF, OUT), lambda n: (0, n, 0)),
            pl.BlockSpec((H, _BNF, OUT), lambda n: (0, n, 0)),
            pl.BlockSpec((1, OUT), lambda n: (0, 0)),
            pl.BlockSpec((C, OUT), lambda n: (0, 0)),
            pl.BlockSpec((1, C), lambda n: (0, 0)),
        ],
        out_specs=pl.BlockSpec((_BNF, C), lambda n: (n, 0)),
        out_shape=jax.ShapeDtypeStruct((N, C), jnp.float32),
    )(acc, den, bias2, W_fc, b_fc2)


def kernel(x, edge_index, exps, exps_c, W_l, b_l, W_r, b_r, att, bias, W_fc, b_fc):
    src = edge_index[:, 0].astype(jnp.int32)
    dst = edge_index[:, 1].astype(jnp.int32)
    xl3, xr3 = _project(x, W_l, b_l.reshape(H, 1, OUT), W_r, b_r.reshape(H, 1, OUT))
    zacc = jnp.zeros((NHP, OUT), jnp.float32)
    zden = jnp.zeros((NHP, OUT), jnp.float32)
    acc, den = _sc_edges(src, dst, att, zacc, zden,
                         xl3[0], xl3[1], xl3[2], xl3[3],
                         xr3[0], xr3[1], xr3[2], xr3[3])
    h = _finalize(acc, den, bias.reshape(1, OUT), W_fc, b_fc.reshape(1, C))
    return (h, exps, exps_c)


# speedup vs baseline: 1.0081x; 1.0081x over previous
import functools

import jax
import jax.numpy as jnp
from jax import lax
from jax.experimental import pallas as pl
from jax.experimental.pallas import tpu as pltpu
from jax.experimental.pallas import tpu_sc as plsc

N = 10000
E = 320000
D = 128
H = 4
OUT = 128
C = 460

NC = 2
NS = 16
NH = N // NC
NHP = NH + 8
EPT = E // NS
EB = 80
NBATCH = EPT // EB
DW = 16


_BN = 2000


def _proj_body(x_ref, wl_ref, bl_ref, wr_ref, br_ref, ol_ref, or_ref):
    xb = x_ref[...]
    dn = (((1,), (1,)), ((), ()))
    l = lax.dot_general(xb, wl_ref[...], dn, preferred_element_type=jnp.float32)
    r = lax.dot_general(xb, wr_ref[...], dn, preferred_element_type=jnp.float32)
    ol_ref[...] = (l + bl_ref[0])[None]
    or_ref[...] = (r + br_ref[0])[None]


def _project(x, W_l, b_l2, W_r, b_r2):
    return pl.pallas_call(
        _proj_body,
        grid=(H, N // _BN),
        in_specs=[
            pl.BlockSpec((_BN, D), lambda h, n: (n, 0)),
            pl.BlockSpec((OUT, D), lambda h, n: (h, 0)),
            pl.BlockSpec((1, 1, OUT), lambda h, n: (h, 0, 0)),
            pl.BlockSpec((OUT, D), lambda h, n: (h, 0)),
            pl.BlockSpec((1, 1, OUT), lambda h, n: (h, 0, 0)),
        ],
        out_specs=[
            pl.BlockSpec((1, _BN, OUT), lambda h, n: (h, n, 0)),
            pl.BlockSpec((1, _BN, OUT), lambda h, n: (h, n, 0)),
        ],
        out_shape=[
            jax.ShapeDtypeStruct((H, N, OUT), jnp.float32),
            jax.ShapeDtypeStruct((H, N, OUT), jnp.float32),
        ],
    )(x, W_l, b_l2, W_r, b_r2)


_sc_mesh = plsc.VectorSubcoreMesh(core_axis_name="c", subcore_axis_name="s")


@functools.partial(
    pl.kernel,
    out_type=[
        jax.ShapeDtypeStruct((H, N, OUT), jnp.float32),
        jax.ShapeDtypeStruct((H, N, OUT), jnp.float32),
    ],
    mesh=_sc_mesh,
    scratch_types=[
        pltpu.VMEM((EB,), jnp.int32),
        pltpu.VMEM((EB,), jnp.int32),
        pltpu.VMEM((EB,), jnp.int32),
        pltpu.VMEM((EB, OUT), jnp.float32),
        pltpu.VMEM((EB, OUT), jnp.float32),
        pltpu.VMEM((EB, OUT), jnp.float32),
        pltpu.VMEM((H, OUT), jnp.float32),
        pltpu.SemaphoreType.DMA,
        pltpu.SemaphoreType.DMA,
        pltpu.VMEM_SHARED((NHP, OUT), jnp.float32),
        pltpu.VMEM_SHARED((NHP, OUT), jnp.float32),
    ],
)
def _sc_edges(src_hbm, dst_hbm, att_hbm, zacc_hbm, zden_hbm,
              xl0, xl1, xl2, xl3, xr0, xr1, xr2, xr3,
              acc_out, den_out,
              srcv, dstv, dstv2, lrows, rrows, wrow, attv,
              sem0, sem1, acc_s, den_s):
    c = lax.axis_index("c")
    s = lax.axis_index("s")
    cbase = c * NH

    pltpu.sync_copy(att_hbm, attv)

    xls = (xl0, xl1, xl2, xl3)
    xrs = (xr0, xr1, xr2, xr3)

    for h in range(H):
        pltpu.sync_copy(zacc_hbm, acc_s)
        pltpu.sync_copy(zden_hbm, den_s)
        plsc.subcore_barrier()

        def _batch_body(b, carry, h=h):
            base = s * EPT + b * EB
            pltpu.sync_copy(src_hbm.at[pl.ds(base, EB)], srcv)
            pltpu.sync_copy(dst_hbm.at[pl.ds(base, EB)], dstv)
            cp_l = pltpu.async_copy(xls[h].at[srcv], lrows, sem0)
            cp_r = pltpu.async_copy(xrs[h].at[dstv], rrows, sem1)
            cp_l.wait()
            cp_r.wait()

            for g in range(EB // 16):
                dv = dstv[pl.ds(g * 16, 16)]
                local = dv - cbase
                oob = (local < 0) | (local >= NH)
                dstv2[pl.ds(g * 16, 16)] = jnp.where(oob, NH, local)

            def _edge_body(e, ecarry, h=h):
                acc = jnp.zeros((16,), jnp.float32)
                for k in range(OUT // 16):
                    lv = lrows[e, pl.ds(k * 16, 16)]
                    rv = rrows[e, pl.ds(k * 16, 16)]
                    v = lv + rv
                    lr = jnp.maximum(v, 0.2 * v)
                    acc = acc + attv[h, pl.ds(k * 16, 16)] * lr
                lane = lax.iota(jnp.int32, 16)
                for sh in (1, 2, 4, 8):
                    acc = acc + acc.at[lane ^ sh].get(mode="promise_in_bounds")
                wv = jnp.exp(acc)
                for k in range(OUT // 16):
                    wrow[e, pl.ds(k * 16, 16)] = wv
                for k in range(OUT // 16):
                    rrows[e, pl.ds(k * 16, 16)] = lrows[e, pl.ds(k * 16, 16)] * wv
                return ecarry

            lax.fori_loop(0, EB, _edge_body, 0)
            pltpu.sync_copy(rrows, acc_s.at[dstv2], add=True)
            pltpu.sync_copy(wrow, den_s.at[dstv2], add=True)
            return carry

        lax.fori_loop(0, NBATCH, _batch_body, 0)
        plsc.subcore_barrier()

        pltpu.sync_copy(acc_s.at[pl.ds(0, NH)],
                        acc_out.at[h, pl.ds(cbase, NH)])
        pltpu.sync_copy(den_s.at[pl.ds(0, NH)],
                        den_out.at[h, pl.ds(cbase, NH)])
        plsc.subcore_barrier()


_BNF = 1000


def _final_body(acc_ref, den_ref, bias_ref, wfc_ref, bfc_ref, out_ref):
    hsum = jnp.zeros((_BNF, OUT), jnp.float32)
    for h in range(H):
        a = acc_ref[h]
        dsum = den_ref[h]
        hsum = hsum + a / (dsum[:, 0:1] + 1e-16)
    hmean = hsum * (1.0 / H) + bias_ref[...]
    dn = (((1,), (1,)), ((), ()))
    out = lax.dot_general(hmean, wfc_ref[...], dn,
                          preferred_element_type=jnp.float32)
    out_ref[...] = out + bfc_ref[...]


def _finalize(acc, den, bias2, W_fc, b_fc2):
    return pl.pallas_call(
        _final_body,
        grid=(N // _BNF,),
        in_specs=[
            pl.BlockSpec((H, _BNF, OUT), lambda n: (0, n, 0)),
            pl.BlockSpec((H, _BNF, OUT), lambda n: (0, n, 0)),
            pl.BlockSpec((1, OUT), lambda n: (0, 0)),
            pl.BlockSpec((C, OUT), lambda n: (0, 0)),
            pl.BlockSpec((1, C), lambda n: (0, 0)),
        ],
        out_specs=pl.BlockSpec((_BNF, C), lambda n: (n, 0)),
        out_shape=jax.ShapeDtypeStruct((N, C), jnp.float32),
    )(acc, den, bias2, W_fc, b_fc2)


def kernel(x, edge_index, exps, exps_c, W_l, b_l, W_r, b_r, att, bias, W_fc, b_fc):
    src = edge_index[:, 0].astype(jnp.int32)
    dst = edge_index[:, 1].astype(jnp.int32)
    xl3, xr3 = _project(x, W_l, b_l.reshape(H, 1, OUT), W_r, b_r.reshape(H, 1, OUT))
    zacc = jnp.zeros((NHP, OUT), jnp.float32)
    zden = jnp.zeros((NHP, OUT), jnp.float32)
    acc, den = _sc_edges(src, dst, att, zacc, zden,
                         xl3[0], xl3[1], xl3[2], xl3[3],
                         xr3[0], xr3[1], xr3[2], xr3[3])
    h = _finalize(acc, den, bias.reshape(1, OUT), W_fc, b_fc.reshape(1, C))
    return (h, exps, exps_c)
